# Initial kernel scaffold; baseline (speedup 1.0000x reference)
import jax
import jax.numpy as jnp
from jax.experimental import pallas as pl


def kernel(positions, cells, emb, W_rad, W_tp, W_eq, Wh1, bh1, Wh2, bh2, species, cell_shifts, center_indices, neighbor_indices, structure_pairs):
    raise NotImplementedError("write your pallas kernel here")


# speedup vs baseline: 13.8392x; 13.8392x over previous
import jax
import jax.numpy as jnp
from jax import lax
from jax.experimental import pallas as pl

N = 10000
E = 320000
K = 32
NMAX = 8
LMAX = 2
CUTOFF = 5.0
NLM = 9

EB = 3200


def _edge_msg_body(vt_ref, src_ref, w2_ref, r9_ref, out_ref):
    x = vt_ref[0, 0, :]
    y = vt_ref[1, 0, :]
    z = vt_ref[2, 0, :]

    r2 = x * x + y * y + z * z + 1e-10
    r = jnp.sqrt(r2)
    inv_r = 1.0 / r
    ux = x * inv_r
    uy = y * inv_r
    uz = z * inv_r

    fc = 0.5 * (jnp.cos((jnp.pi / CUTOFF) * jnp.minimum(r, CUTOFF)) + 1.0)
    s1 = jnp.sin((jnp.pi / CUTOFF) * r)
    c1 = jnp.cos((jnp.pi / CUTOFF) * r)
    two_c1 = 2.0 * c1
    g = inv_r * fc

    rbs = [s1 * g]
    s_prev, s_cur = jnp.zeros_like(s1), s1
    for _ in range(1, NMAX):
        s_prev, s_cur = s_cur, two_c1 * s_cur - s_prev
        rbs.append(s_cur * g)
    rbst = jnp.stack(rbs)

    d1 = 0.4886025119029199
    d2 = 1.0925484305920792
    sht = jnp.stack([
        jnp.full_like(x, 0.28209479177387814),
        d1 * uy, d1 * uz, d1 * ux,
        d2 * ux * uy, d2 * uy * uz,
        0.31539156525252005 * (3.0 * uz * uz - 1.0),
        d2 * ux * uz,
        0.5462742152960396 * (ux * ux - uy * uy),
    ])

    dn = (((0,), (0,)), ((), ()))
    rad = lax.dot_general(rbst, w2_ref[:], dn,
                          preferred_element_type=jnp.float32)
    sh2 = lax.dot_general(sht, r9_ref[:], dn,
                          preferred_element_type=jnp.float32)
    src = src_ref[:]
    src9 = jnp.concatenate([src] * NLM, axis=1)
    out_ref[:] = rad * sh2 * src9


_edge_msg = pl.pallas_call(
    _edge_msg_body,
    grid=(E // EB,),
    in_specs=[
        pl.BlockSpec((3, 1, EB), lambda i: (0, 0, i)),
        pl.BlockSpec((EB, K), lambda i: (i, 0)),
        pl.BlockSpec((NMAX, NLM * K), lambda i: (0, 0)),
        pl.BlockSpec((NLM, NLM * K), lambda i: (0, 0)),
    ],
    out_specs=pl.BlockSpec((EB, NLM * K), lambda i: (i, 0)),
    out_shape=jax.ShapeDtypeStruct((E, NLM * K), jnp.float32),
)


def _expansion_mats(W_rad):
    l_of_lm = [0, 1, 1, 1, 2, 2, 2, 2, 2]
    w2 = jnp.concatenate([W_rad[l_of_lm[lm]] for lm in range(NLM)], axis=1)
    r9 = jnp.zeros((NLM, NLM * K), jnp.float32)
    for lm in range(NLM):
        r9 = r9.at[lm, lm * K:(lm + 1) * K].set(1.0)
    return w2, r9


def _tp_contract(feats_flat, W_tp):
    f = feats_flat.reshape(N, NLM, K)
    inv = f[:, 0, :]
    sq0 = f[:, 0, :] * f[:, 0, :]
    sq1 = jnp.sum(f[:, 1:4, :] * f[:, 1:4, :], axis=1)
    sq2 = jnp.sum(f[:, 4:9, :] * f[:, 4:9, :], axis=1)
    return inv + sq0 @ W_tp[0] + sq1 @ W_tp[1] + sq2 @ W_tp[2]


def kernel(positions, cells, emb, W_rad, W_tp, W_eq, Wh1, bh1, Wh2, bh2,
           species, cell_shifts, center_indices, neighbor_indices,
           structure_pairs):
    shift = jnp.einsum('ek,ekl->el', cell_shifts.astype(positions.dtype),
                       cells[structure_pairs])
    vec = positions[neighbor_indices] - positions[center_indices] + shift
    vecT = vec.T.reshape(3, 1, E)

    w2, r9 = _expansion_mats(W_rad)

    src1 = emb[species[neighbor_indices]]
    msg1 = _edge_msg(vecT, src1, w2, r9)
    f1 = jax.ops.segment_sum(msg1, center_indices, num_segments=N)
    inv = _tp_contract(f1, W_tp)

    src2 = inv[neighbor_indices]
    msg2 = _edge_msg(vecT, src2, w2, r9)
    f2 = jax.ops.segment_sum(msg2, center_indices, num_segments=N)
    f2 = f2.reshape(N, NLM, K)
    g0 = f2[:, 0:1, :] @ W_eq[0]
    g1 = f2[:, 1:4, :] @ W_eq[1]
    g2 = f2[:, 4:9, :] @ W_eq[2]
    inv2 = g0[:, 0, :]
    inv2 = inv2 + jnp.sum(g0 * g0, axis=1) @ W_tp[0]
    inv2 = inv2 + jnp.sum(g1 * g1, axis=1) @ W_tp[1]
    inv2 = inv2 + jnp.sum(g2 * g2, axis=1) @ W_tp[2]

    h = jnp.tanh(inv2 @ Wh1 + bh1)
    return h @ Wh2 + bh2


# speedup vs baseline: 36.1986x; 2.6157x over previous
import functools

import jax
import jax.numpy as jnp
from jax import lax
from jax.experimental import pallas as pl
from jax.experimental.pallas import tpu as pltpu
from jax.experimental.pallas import tpu_sc as plsc

N = 10000
E = 320000
S = 32
K = 32
NMAX = 8
LMAX = 2
CUTOFF = 5.0
NLM = 9

EB = 3200

_SC_INFO = plsc.get_sparse_core_info()
NC = _SC_INFO.num_cores
NS = _SC_INFO.num_subcores
NW = NC * NS
EPT = E // NW
GC = 80
_SC_MESH = plsc.VectorSubcoreMesh(core_axis_name="c", subcore_axis_name="s")


def _make_row_gather(ncols):
    @functools.partial(
        pl.kernel, mesh=_SC_MESH,
        out_type=jax.ShapeDtypeStruct((E, ncols), jnp.float32),
        scratch_types=[
            pltpu.VMEM((GC,), jnp.int32),
            pltpu.VMEM((GC, ncols), jnp.float32),
            pltpu.SemaphoreType.DMA,
        ],
    )
    def _gather(table_hbm, idx_hbm, out_hbm, idx_v, rows_v, sem):
        wid = lax.axis_index("s") * NC + lax.axis_index("c")
        base = wid * EPT

        def chunk(j, _):
            e0 = base + j * GC
            pltpu.sync_copy(idx_hbm.at[pl.ds(e0, GC)], idx_v)
            pltpu.async_copy(table_hbm.at[idx_v], rows_v, sem).wait()
            pltpu.sync_copy(rows_v, out_hbm.at[pl.ds(e0, GC)])
            return 0

        lax.fori_loop(0, EPT // GC, chunk, 0)

    return _gather


TW = 128
_row_gather128 = _make_row_gather(TW)


def _edge_geom(x, y, z):
    r2 = x * x + y * y + z * z + 1e-10
    r = jnp.sqrt(r2)
    inv_r = 1.0 / r
    ux = x * inv_r
    uy = y * inv_r
    uz = z * inv_r

    fc = 0.5 * (jnp.cos((jnp.pi / CUTOFF) * jnp.minimum(r, CUTOFF)) + 1.0)
    s1 = jnp.sin((jnp.pi / CUTOFF) * r)
    c1 = jnp.cos((jnp.pi / CUTOFF) * r)
    two_c1 = 2.0 * c1
    g = inv_r * fc

    rbs = [s1 * g]
    s_prev, s_cur = jnp.zeros_like(s1), s1
    for _ in range(1, NMAX):
        s_prev, s_cur = s_cur, two_c1 * s_cur - s_prev
        rbs.append(s_cur * g)
    rbst = jnp.stack(rbs)

    d1 = 0.4886025119029199
    d2 = 1.0925484305920792
    sht = jnp.stack([
        jnp.full_like(x, 0.28209479177387814),
        d1 * uy, d1 * uz, d1 * ux,
        d2 * ux * uy, d2 * uy * uz,
        0.31539156525252005 * (3.0 * uz * uz - 1.0),
        d2 * ux * uz,
        0.5462742152960396 * (ux * ux - uy * uy),
    ])
    return rbst, sht


_DN0 = (((0,), (0,)), ((), ()))
_DN1 = (((1,), (1,)), ((), ()))


def _edge_vec(gn_ref, gc_ref, sp_ref, cs0_ref, cs1_ref, cs2_ref, crp_ref):
    eye = (lax.broadcasted_iota(jnp.int32, (8, TW), 0)
           == lax.broadcasted_iota(jnp.int32, (8, TW), 1)).astype(jnp.float32)
    gnt = lax.dot_general(eye, gn_ref[:], _DN1,
                          preferred_element_type=jnp.float32)
    gct = lax.dot_general(eye, gc_ref[:], _DN1,
                          preferred_element_type=jnp.float32)
    sp = sp_ref[0, 0, :]
    cs0 = cs0_ref[0, 0, :]
    cs1 = cs1_ref[0, 0, :]
    cs2 = cs2_ref[0, 0, :]
    iota96 = lax.broadcasted_iota(jnp.int32, (3 * S, EB), 0)
    sp3 = 3 * sp[None, :]
    mt = ((iota96 == sp3).astype(jnp.float32) * cs0[None, :]
          + (iota96 == sp3 + 1).astype(jnp.float32) * cs1[None, :]
          + (iota96 == sp3 + 2).astype(jnp.float32) * cs2[None, :])
    shift = lax.dot_general(crp_ref[:], mt, _DN0,
                            preferred_element_type=jnp.float32)
    vx = gnt[0] - gct[0] + shift[0]
    vy = gnt[1] - gct[1] + shift[1]
    vz = gnt[2] - gct[2] + shift[2]
    return vx, vy, vz, gnt[3]


def _edge_msg1_body(gn_ref, gc_ref, sp_ref, cs0_ref, cs1_ref, cs2_ref,
                    crp_ref, w2_ref, r9_ref, emb9_ref, msg_ref, geom_ref):
    vx, vy, vz, spec = _edge_vec(gn_ref, gc_ref, sp_ref, cs0_ref, cs1_ref,
                                 cs2_ref, crp_ref)
    rbst, sht = _edge_geom(vx, vy, vz)
    oh = jnp.stack([(spec == float(s2)).astype(jnp.float32)
                    for s2 in range(4)])
    rad = lax.dot_general(rbst, w2_ref[:], _DN0,
                          preferred_element_type=jnp.float32)
    sh2 = lax.dot_general(sht, r9_ref[:], _DN0,
                          preferred_element_type=jnp.float32)
    src9 = lax.dot_general(oh, emb9_ref[:], _DN0,
                           preferred_element_type=jnp.float32)
    msg_ref[:] = rad * sh2 * src9
    zero = jnp.zeros_like(vx)
    geom_ref[:] = jnp.stack([vx, vy, vz, spec, zero, zero, zero, zero])


def _edge_msg2_body(geom_ref, src_ref, w2_ref, r9_ref, out_ref):
    vx = geom_ref[0, :]
    vy = geom_ref[1, :]
    vz = geom_ref[2, :]
    rbst, sht = _edge_geom(vx, vy, vz)
    rad = lax.dot_general(rbst, w2_ref[:], _DN0,
                          preferred_element_type=jnp.float32)
    sh2 = lax.dot_general(sht, r9_ref[:], _DN0,
                          preferred_element_type=jnp.float32)
    src9 = jnp.concatenate([src_ref[:, :K]] * NLM, axis=1)
    out_ref[:] = rad * sh2 * src9


def _lane(i):
    return (0, 0, i)


_edge_msg1 = pl.pallas_call(
    _edge_msg1_body,
    grid=(E // EB,),
    in_specs=[
        pl.BlockSpec((EB, TW), lambda i: (i, 0)),
        pl.BlockSpec((EB, TW), lambda i: (i, 0)),
        pl.BlockSpec((1, 1, EB), _lane),
        pl.BlockSpec((1, 1, EB), _lane),
        pl.BlockSpec((1, 1, EB), _lane),
        pl.BlockSpec((1, 1, EB), _lane),
        pl.BlockSpec((3 * S, 128), lambda i: (0, 0)),
        pl.BlockSpec((NMAX, NLM * K), lambda i: (0, 0)),
        pl.BlockSpec((NLM, NLM * K), lambda i: (0, 0)),
        pl.BlockSpec((4, NLM * K), lambda i: (0, 0)),
    ],
    out_specs=[
        pl.BlockSpec((EB, NLM * K), lambda i: (i, 0)),
        pl.BlockSpec((8, EB), lambda i: (0, i)),
    ],
    out_shape=[
        jax.ShapeDtypeStruct((E, NLM * K), jnp.float32),
        jax.ShapeDtypeStruct((8, E), jnp.float32),
    ],
)

_edge_msg2 = pl.pallas_call(
    _edge_msg2_body,
    grid=(E // EB,),
    in_specs=[
        pl.BlockSpec((8, EB), lambda i: (0, i)),
        pl.BlockSpec((EB, TW), lambda i: (i, 0)),
        pl.BlockSpec((NMAX, NLM * K), lambda i: (0, 0)),
        pl.BlockSpec((NLM, NLM * K), lambda i: (0, 0)),
    ],
    out_specs=pl.BlockSpec((EB, NLM * K), lambda i: (i, 0)),
    out_shape=jax.ShapeDtypeStruct((E, NLM * K), jnp.float32),
)


def _expansion_mats(W_rad):
    l_of_lm = [0, 1, 1, 1, 2, 2, 2, 2, 2]
    w2 = jnp.concatenate([W_rad[l_of_lm[lm]] for lm in range(NLM)], axis=1)
    r9 = jnp.zeros((NLM, NLM * K), jnp.float32)
    for lm in range(NLM):
        r9 = r9.at[lm, lm * K:(lm + 1) * K].set(1.0)
    return w2, r9


def _tp_contract(feats_flat, W_tp):
    f = feats_flat.reshape(N, NLM, K)
    inv = f[:, 0, :]
    sq0 = f[:, 0, :] * f[:, 0, :]
    sq1 = jnp.sum(f[:, 1:4, :] * f[:, 1:4, :], axis=1)
    sq2 = jnp.sum(f[:, 4:9, :] * f[:, 4:9, :], axis=1)
    return inv + sq0 @ W_tp[0] + sq1 @ W_tp[1] + sq2 @ W_tp[2]


def kernel(positions, cells, emb, W_rad, W_tp, W_eq, Wh1, bh1, Wh2, bh2,
           species, cell_shifts, center_indices, neighbor_indices,
           structure_pairs):
    nbr = neighbor_indices.astype(jnp.int32)
    ctr = center_indices.astype(jnp.int32)
    sp = structure_pairs.astype(jnp.int32).reshape(1, 1, E)
    csf = cell_shifts.astype(jnp.float32)
    cs0 = csf[:, 0].reshape(1, 1, E)
    cs1 = csf[:, 1].reshape(1, 1, E)
    cs2 = csf[:, 2].reshape(1, 1, E)

    ptab = (jnp.zeros((N, TW), jnp.float32)
            .at[:, :3].set(positions)
            .at[:, 3].set(species.astype(jnp.float32)))
    crp = jnp.zeros((3 * S, 128), jnp.float32).at[:, :3].set(
        cells.reshape(3 * S, 3))

    gn = _row_gather128(ptab, nbr)
    gc = _row_gather128(ptab, ctr)

    w2, r9 = _expansion_mats(W_rad)
    emb9 = jnp.concatenate([emb] * NLM, axis=1)

    msg1, geom = _edge_msg1(gn, gc, sp, cs0, cs1, cs2, crp, w2, r9, emb9)
    f1 = jax.ops.segment_sum(msg1, center_indices, num_segments=N)
    inv = _tp_contract(f1, W_tp)

    invp = jnp.zeros((N, TW), jnp.float32).at[:, :K].set(inv)
    src2 = _row_gather128(invp, nbr)
    msg2 = _edge_msg2(geom, src2, w2, r9)
    f2 = jax.ops.segment_sum(msg2, center_indices, num_segments=N)
    f2 = f2.reshape(N, NLM, K)
    g0 = f2[:, 0:1, :] @ W_eq[0]
    g1 = f2[:, 1:4, :] @ W_eq[1]
    g2 = f2[:, 4:9, :] @ W_eq[2]
    inv2 = g0[:, 0, :]
    inv2 = inv2 + jnp.sum(g0 * g0, axis=1) @ W_tp[0]
    inv2 = inv2 + jnp.sum(g1 * g1, axis=1) @ W_tp[1]
    inv2 = inv2 + jnp.sum(g2 * g2, axis=1) @ W_tp[2]

    h = jnp.tanh(inv2 @ Wh1 + bh1)
    return h @ Wh2 + bh2
